# Initial kernel scaffold; baseline (speedup 1.0000x reference)
import functools
import math

import jax
import jax.numpy as jnp
from jax import lax
from jax.experimental import pallas as pl
from jax.experimental.pallas import tpu as pltpu
from jax.experimental.pallas import tpu_sc as plsc

TWO_PI = jnp.float32(2.0 * math.pi)

_B = 16384
_K = 1000

_NC = 2
_NS = 16
_NW = _NC * _NS
_BPW = _B // _NW
_LANES = 16
_VREGS = _BPW // _LANES

_BM = 1024


def _sc_phi_body(x1_hbm, x2_hbm, ip_hbm, out_hbm, tab_v, i1_v, i2_v, phi_v):
    wid = lax.axis_index("s") * _NC + lax.axis_index("c")
    base = wid * _BPW
    pltpu.sync_copy(ip_hbm, tab_v)
    pltpu.sync_copy(x1_hbm.at[pl.ds(base, _BPW)], i1_v)
    pltpu.sync_copy(x2_hbm.at[pl.ds(base, _BPW)], i2_v)

    def step(i, carry):
        sl = pl.ds(i * _LANES, _LANES)
        p1 = plsc.load_gather(tab_v, [i1_v[sl]])
        p2 = plsc.load_gather(tab_v, [i2_v[sl]])
        s = p1 + p2
        phi_v[sl] = jnp.where(s >= TWO_PI, s - TWO_PI, s)
        return carry

    lax.fori_loop(0, _VREGS, step, 0)
    pltpu.sync_copy(phi_v, out_hbm.at[pl.ds(base, _BPW)])


_sc_phi = functools.partial(
    pl.kernel,
    mesh=plsc.VectorSubcoreMesh(core_axis_name="c", subcore_axis_name="s"),
    out_type=jax.ShapeDtypeStruct((_B,), jnp.float32),
    scratch_types=[
        pltpu.VMEM((_K,), jnp.float32),
        pltpu.VMEM((_BPW,), jnp.int32),
        pltpu.VMEM((_BPW,), jnp.int32),
        pltpu.VMEM((_BPW,), jnp.float32),
    ],
)(_sc_phi_body)


def _tc_logits_body(phi_ref, op_ref, out_ref):
    d = jnp.abs(phi_ref[...] - op_ref[...])
    out_ref[...] = jnp.maximum(-d, d - TWO_PI)


def _tc_logits(phi, output_phases):
    phi2 = phi.reshape(_B, 1)
    op2 = output_phases.reshape(1, _K)
    return pl.pallas_call(
        _tc_logits_body,
        grid=(_B // _BM,),
        in_specs=[
            pl.BlockSpec((_BM, 1), lambda i: (i, 0)),
            pl.BlockSpec((1, _K), lambda i: (0, 0)),
        ],
        out_specs=pl.BlockSpec((_BM, _K), lambda i: (i, 0)),
        out_shape=jax.ShapeDtypeStruct((_B, _K), jnp.float32),
    )(phi2, op2)


@jax.jit
def kernel(x1, x2, input_phases, output_phases):
    phi = _sc_phi(x1, x2, input_phases)
    return _tc_logits(phi, output_phases)


# speedup vs baseline: 2.0685x; 2.0685x over previous
import functools
import math

import jax
import jax.numpy as jnp
from jax import lax
from jax.experimental import pallas as pl
from jax.experimental.pallas import tpu as pltpu
from jax.experimental.pallas import tpu_sc as plsc

TWO_PI = 2.0 * math.pi

_B = 16384
_K = 1000

_NC = 2
_NS = 16
_NW = _NC * _NS
_BPW = _B // _NW
_LANES = 16
_VREGS = _BPW // _LANES

_BM = 1024


def _sc_phi_body(x1_hbm, x2_hbm, ip_hbm, out_hbm, tab_v, i1_v, i2_v, phi_v):
    wid = lax.axis_index("s") * _NC + lax.axis_index("c")
    base = wid * _BPW
    pltpu.sync_copy(ip_hbm, tab_v)
    pltpu.sync_copy(x1_hbm.at[pl.ds(base, _BPW)], i1_v)
    pltpu.sync_copy(x2_hbm.at[pl.ds(base, _BPW)], i2_v)

    def step(i, carry):
        sl = pl.ds(i * _LANES, _LANES)
        p1 = plsc.load_gather(tab_v, [i1_v[sl]])
        p2 = plsc.load_gather(tab_v, [i2_v[sl]])
        s = p1 + p2
        phi_v[sl] = jnp.where(s >= TWO_PI, s - TWO_PI, s)
        return carry

    lax.fori_loop(0, _VREGS, step, 0)
    pltpu.sync_copy(phi_v, out_hbm.at[pl.ds(base, _BPW)])


_sc_phi = functools.partial(
    pl.kernel,
    mesh=plsc.VectorSubcoreMesh(core_axis_name="c", subcore_axis_name="s"),
    out_type=jax.ShapeDtypeStruct((_B,), jnp.float32),
    scratch_types=[
        pltpu.VMEM((_K,), jnp.float32),
        pltpu.VMEM((_BPW,), jnp.int32),
        pltpu.VMEM((_BPW,), jnp.int32),
        pltpu.VMEM((_BPW,), jnp.float32),
    ],
    compiler_params=pltpu.CompilerParams(needs_layout_passes=False),
)(_sc_phi_body)


def _tc_logits_body(phi_ref, op_ref, out_ref):
    d = jnp.abs(phi_ref[...] - op_ref[...])
    out_ref[...] = jnp.maximum(-d, d - TWO_PI)


def _tc_logits(phi, output_phases):
    phi2 = phi.reshape(_B, 1)
    op2 = output_phases.reshape(1, _K)
    return pl.pallas_call(
        _tc_logits_body,
        grid=(_B // _BM,),
        in_specs=[
            pl.BlockSpec((_BM, 1), lambda i: (i, 0)),
            pl.BlockSpec((1, _K), lambda i: (0, 0)),
        ],
        out_specs=pl.BlockSpec((_BM, _K), lambda i: (i, 0)),
        out_shape=jax.ShapeDtypeStruct((_B, _K), jnp.float32),
    )(phi2, op2)


@jax.jit
def kernel(x1, x2, input_phases, output_phases):
    phi = _sc_phi(x1, x2, input_phases)
    return _tc_logits(phi, output_phases)


# speedup vs baseline: 2.1141x; 1.0220x over previous
import functools
import math

import jax
import jax.numpy as jnp
from jax import lax
from jax.experimental import pallas as pl
from jax.experimental.pallas import tpu as pltpu
from jax.experimental.pallas import tpu_sc as plsc

TWO_PI = 2.0 * math.pi

_B = 16384
_K = 1000

_NC = 2
_NS = 16
_NW = _NC * _NS
_BPW = _B // _NW
_LANES = 16
_VREGS = _BPW // _LANES

_BM = 4096


def _sc_phi_body(x1_hbm, x2_hbm, ip_hbm, out_hbm, tab_v, i1_v, i2_v, phi_v):
    wid = lax.axis_index("s") * _NC + lax.axis_index("c")
    base = wid * _BPW
    pltpu.sync_copy(ip_hbm, tab_v)
    pltpu.sync_copy(x1_hbm.at[pl.ds(base, _BPW)], i1_v)
    pltpu.sync_copy(x2_hbm.at[pl.ds(base, _BPW)], i2_v)

    def step(i, carry):
        sl = pl.ds(i * _LANES, _LANES)
        p1 = plsc.load_gather(tab_v, [i1_v[sl]])
        p2 = plsc.load_gather(tab_v, [i2_v[sl]])
        s = p1 + p2
        phi_v[sl] = jnp.where(s >= TWO_PI, s - TWO_PI, s)
        return carry

    lax.fori_loop(0, _VREGS, step, 0)
    pltpu.sync_copy(phi_v, out_hbm.at[pl.ds(base, _BPW)])


_sc_phi = functools.partial(
    pl.kernel,
    mesh=plsc.VectorSubcoreMesh(core_axis_name="c", subcore_axis_name="s"),
    out_type=jax.ShapeDtypeStruct((_B,), jnp.float32),
    scratch_types=[
        pltpu.VMEM((_K,), jnp.float32),
        pltpu.VMEM((_BPW,), jnp.int32),
        pltpu.VMEM((_BPW,), jnp.int32),
        pltpu.VMEM((_BPW,), jnp.float32),
    ],
    compiler_params=pltpu.CompilerParams(needs_layout_passes=False),
)(_sc_phi_body)


def _tc_logits_body(phi_ref, op_ref, out_ref):
    d = jnp.abs(phi_ref[...] - op_ref[...])
    out_ref[...] = jnp.maximum(-d, d - TWO_PI)


def _tc_logits(phi, output_phases):
    phi2 = phi.reshape(_B, 1)
    op2 = output_phases.reshape(1, _K)
    return pl.pallas_call(
        _tc_logits_body,
        grid=(_B // _BM,),
        in_specs=[
            pl.BlockSpec((_BM, 1), lambda i: (i, 0)),
            pl.BlockSpec((1, _K), lambda i: (0, 0)),
        ],
        out_specs=pl.BlockSpec((_BM, _K), lambda i: (i, 0)),
        out_shape=jax.ShapeDtypeStruct((_B, _K), jnp.float32),
    )(phi2, op2)


_NBUF = 8
_CHUNK = 1024
_NCHUNKS = _B // _CHUNK


def _tc_manual_body(phi_ref, op_ref, out_ref, scratch, sems):
    op = op_ref[...]
    for step in range(_NCHUNKS):
        buf = step % _NBUF
        rows = pl.ds(step * _CHUNK, _CHUNK)
        if step >= _NBUF:
            pltpu.make_async_copy(
                scratch.at[buf],
                out_ref.at[pl.ds((step - _NBUF) * _CHUNK, _CHUNK), :],
                sems.at[buf],
            ).wait()
        d = jnp.abs(phi_ref[rows, :] - op)
        scratch[buf] = jnp.maximum(-d, d - TWO_PI)
        pltpu.make_async_copy(
            scratch.at[buf], out_ref.at[rows, :], sems.at[buf]
        ).start()
    for step in range(_NCHUNKS - _NBUF, _NCHUNKS):
        buf = step % _NBUF
        pltpu.make_async_copy(
            scratch.at[buf],
            out_ref.at[pl.ds(step * _CHUNK, _CHUNK), :],
            sems.at[buf],
        ).wait()


def _tc_logits_manual(phi, output_phases):
    phi2 = phi.reshape(_B, 1)
    op2 = output_phases.reshape(1, _K)
    return pl.pallas_call(
        _tc_manual_body,
        in_specs=[
            pl.BlockSpec(memory_space=pltpu.VMEM),
            pl.BlockSpec(memory_space=pltpu.VMEM),
        ],
        out_specs=pl.BlockSpec(memory_space=pl.ANY),
        out_shape=jax.ShapeDtypeStruct((_B, _K), jnp.float32),
        scratch_shapes=[
            pltpu.VMEM((_NBUF, _CHUNK, _K), jnp.float32),
            pltpu.SemaphoreType.DMA((_NBUF,)),
        ],
    )(phi2, op2)


@jax.jit
def kernel(x1, x2, input_phases, output_phases):
    phi = _sc_phi(x1, x2, input_phases)
    return _tc_logits_manual(phi, output_phases)
